# baseline (speedup 1.0000x reference)
import jax
import jax.numpy as jnp
from jax import lax
from jax.experimental import pallas as pl
from jax.experimental.pallas import tpu as pltpu
from jax.experimental.pallas import tpu_sc as plsc

_L = 17
_NC = 153
_MODE_OFFSETS = (
    ((0, 0), (0, 1), (1, 0), (1, 1)),
    ((0, 0), (0, 2), (2, 0), (2, 2)),
    ((0, 0), (1, 1), (1, 2), (2, 1)),
)

_NUM_CORES = 2
_NUM_SUBCORES = 16
_NW = _NUM_CORES * _NUM_SUBCORES

_B, _H, _W = 8, 512, 512
_WP = 520
_ROWS_PER_TILE = (_B * _H) // _NW
_TILES_PER_IMG = _H // _ROWS_PER_TILE
_R = 4
_CHUNKS = _ROWS_PER_TILE // _R
_MSTRIDE = 272
_MSB_WORDS = _NC * _MSTRIDE
_LSB_WORDS = 16 ** 4

def _bf(w):
    return plsc.bitcast(w, jnp.bfloat16)


def _body(xp_ref, t256_ref, tmsb_ref, tlsb_ref, out_ref,
          t256_v, tbl_v, in_v, pv_v, o_v, in_sem, pv_sem, wr_sem, tb_sem):
    wid = lax.axis_index("s") * _NUM_CORES + lax.axis_index("c")
    b = wid // _TILES_PER_IMG
    h0 = (wid % _TILES_PER_IMG) * _ROWS_PER_TILE

    tmsb_v = tbl_v.at[pl.ds(0, _MSB_WORDS)]
    tlsb_v = tbl_v.at[pl.ds(_MSB_WORDS, _LSB_WORDS)]

    def start_tbl(m):
        pltpu.async_copy(tmsb_ref.at[m], tmsb_v, tb_sem)
        pltpu.async_copy(tlsb_ref.at[m], tlsb_v, tb_sem)

    def wait_tbl(m):
        pltpu.make_async_copy(tmsb_ref.at[m], tmsb_v, tb_sem).wait()
        pltpu.make_async_copy(tlsb_ref.at[m], tlsb_v, tb_sem).wait()

    start_tbl(0)
    pltpu.sync_copy(t256_ref, t256_v)

    def in_src(ci):
        return xp_ref.at[b, pl.ds(h0 + ci * _R, _R + 2)]

    def pv_src(ci):
        return out_ref.at[b, 0, pl.ds(h0 + ci * _R, _R)]

    def wr_pair(ci, x, m):
        if m < 2:
            return o_v[x].at[0], pv_src(ci)
        return o_v[x], out_ref.at[b, :, pl.ds(h0 + ci * _R, _R)]

    def start_in(ci, x):
        pltpu.async_copy(in_src(ci), in_v[x], in_sem[x])

    def start_pv(ci, x):
        pltpu.async_copy(pv_src(ci), pv_v[x], pv_sem[x])

    for m in range(3):
        offs = _MODE_OFFSETS[m]

        def step(ci, x, wr_wait_mode, m=m, offs=offs):
            y = 1 - x
            iv, pv, ov = in_v[x], pv_v[x], o_v[x]
            wsrc, wdst = wr_pair(ci, x, m)
            pltpu.make_async_copy(in_src(ci), iv, in_sem[x]).wait()
            if m > 0:
                pltpu.make_async_copy(pv_src(ci), pv, pv_sem[x]).wait()
            if wr_wait_mode is not None:
                pltpu.make_async_copy(
                    *wr_pair(ci, x, wr_wait_mode), wr_sem[x]).wait()
            if m > 0:
                if isinstance(ci, int):
                    if ci + 1 < _CHUNKS:
                        start_pv(ci + 1, y)
                else:
                    @pl.when(ci < _CHUNKS - 1)
                    def _():
                        start_pv(ci + 1, y)

            @plsc.parallel_loop(0, _R * (_W // 16), 1, unroll=2)
            def it(i):
                r = i >> 5
                x0 = (i & 31) << 4
                p = [iv[r + dy, pl.ds(x0 + dx, 16)] for (dy, dx) in offs]
                ip = [q.astype(jnp.int32) for q in p]
                idx01 = (ip[0] & 0xF0) | (ip[1] >> 4)
                base12 = plsc.load_gather(t256_v, [idx01])
                msb3 = ip[3] >> 4
                flat = base12 + (ip[2] & 0xF0) + (ip[2] >> 4) + msb3
                wv = plsc.load_gather(tmsb_v, [flat])
                wvh = plsc.load_gather(tmsb_v, [flat + 1])
                idxl = (((ip[0] & 15) << 12) | ((ip[1] & 15) << 8)
                        | ((ip[2] & 15) << 4) | (ip[3] & 15))
                wl = plsc.load_gather(tlsb_v, [idxl])
                frac = p[3] * 0.0625 - msb3.astype(jnp.float32)
                fb = plsc.pack(frac, frac, format=plsc.PackFormat.INTERLEAVED)
                vb = _bf(wv)
                t = vb + (_bf(wvh) - vb) * fb + _bf(wl)
                if m > 0:
                    t = t + _bf(pv[r, pl.ds(x0, 16)])
                if m < 2:
                    ov[0, r, pl.ds(x0, 16)] = plsc.bitcast(t, jnp.float32)
                else:
                    c0, c1 = plsc.unpack(
                        t, format=plsc.PackFormat.INTERLEAVED)
                    ov[0, r, pl.ds(x0, 16)] = c0
                    ov[1, r, pl.ds(x0, 16)] = c1

            pltpu.async_copy(wsrc, wdst, wr_sem[x])
            if isinstance(ci, int):
                if ci + 2 < _CHUNKS:
                    start_in(ci + 2, x)
            else:
                @pl.when(ci < _CHUNKS - 2)
                def _():
                    start_in(ci + 2, x)

        start_in(0, 0)
        start_in(1, 1)
        if m > 0:
            start_pv(0, 0)
        wait_tbl(m)
        step(0, 0, wr_wait_mode=None if m == 0 else m - 1)
        step(1, 1, wr_wait_mode=None if m == 0 else m - 1)

        def pair(k, _):
            step(2 * k, 0, wr_wait_mode=m)
            step(2 * k + 1, 1, wr_wait_mode=m)
            return 0

        lax.fori_loop(1, _CHUNKS // 2, pair, 0)
        if m < 2:
            start_tbl(m + 1)
    pltpu.make_async_copy(*wr_pair(0, 0, 2), wr_sem[0]).wait()
    pltpu.make_async_copy(*wr_pair(1, 1, 2), wr_sem[1]).wait()


def _pack_pairs(t2):
    bits = jax.lax.bitcast_convert_type(
        t2.astype(jnp.bfloat16), jnp.uint16).astype(jnp.uint32)
    w = bits[..., 0] | (bits[..., 1] << 16)
    return jax.lax.bitcast_convert_type(w, jnp.int32)


@jax.jit
def kernel(x, lut_msb, lut_lsb, ref2index):
    xp = jnp.pad(x[:, 0], ((0, 0), (0, 2), (0, _WP - _W)), mode="edge")
    tmsb = _pack_pairs(
        lut_msb.reshape(3, _NC, _L * _L, 2)[:, :, :_MSTRIDE, :] * (1.0 / 3.0)
    ).reshape(3, _MSB_WORDS)
    tlsb = _pack_pairs(lut_lsb * (1.0 / 3.0))
    t256 = (ref2index[:16, :16].reshape(-1) * _MSTRIDE).astype(jnp.int32)

    mesh = plsc.VectorSubcoreMesh(core_axis_name="c", subcore_axis_name="s")
    run = pl.kernel(
        _body,
        out_type=jax.ShapeDtypeStruct((_B, 2, _H, _W), jnp.float32),
        mesh=mesh,
        compiler_params=pltpu.CompilerParams(
            use_tc_tiling_on_sc=False, needs_layout_passes=False),
        scratch_types=[
            pltpu.VMEM((256,), jnp.int32),
            pltpu.VMEM((_MSB_WORDS + _LSB_WORDS,), jnp.int32),
            [pltpu.VMEM((_R + 2, _WP), jnp.float32) for _ in range(2)],
            [pltpu.VMEM((_R, _W), jnp.float32) for _ in range(2)],
            [pltpu.VMEM((2, _R, _W), jnp.float32) for _ in range(2)],
            [pltpu.SemaphoreType.DMA for _ in range(2)],
            [pltpu.SemaphoreType.DMA for _ in range(2)],
            [pltpu.SemaphoreType.DMA for _ in range(2)],
            pltpu.SemaphoreType.DMA,
        ],
    )
    return run(xp, t256, tmsb, tlsb)


# speedup vs baseline: 1.0021x; 1.0021x over previous
import jax
import jax.numpy as jnp
from jax import lax
from jax.experimental import pallas as pl
from jax.experimental.pallas import tpu as pltpu
from jax.experimental.pallas import tpu_sc as plsc

_L = 17
_NC = 153
_MODE_OFFSETS = (
    ((0, 0), (0, 1), (1, 0), (1, 1)),
    ((0, 0), (0, 2), (2, 0), (2, 2)),
    ((0, 0), (1, 1), (1, 2), (2, 1)),
)

_NUM_CORES = 2
_NUM_SUBCORES = 16
_NW = _NUM_CORES * _NUM_SUBCORES

_B, _H, _W = 8, 512, 512
_WP = 520
_ROWS_PER_TILE = (_B * _H) // _NW
_TILES_PER_IMG = _H // _ROWS_PER_TILE
_R = 4
_CHUNKS = _ROWS_PER_TILE // _R
_MSTRIDE = 272
_MSB_WORDS = _NC * _MSTRIDE
_LSB_WORDS = 16 ** 4

def _bf(w):
    return plsc.bitcast(w, jnp.bfloat16)


def _body(xp_ref, t256_ref, tmsb_ref, tlsb_ref, out_ref,
          t256_v, tbl_v, in_v, pv_v, o_v, in_sem, pv_sem, wr_sem, tb_sem):
    wid = lax.axis_index("s") * _NUM_CORES + lax.axis_index("c")
    b = wid // _TILES_PER_IMG
    h0 = (wid % _TILES_PER_IMG) * _ROWS_PER_TILE

    tmsb_v = tbl_v.at[pl.ds(0, _MSB_WORDS)]
    tlsb_v = tbl_v.at[pl.ds(_MSB_WORDS, _LSB_WORDS)]

    def start_tbl(m):
        pltpu.async_copy(tmsb_ref.at[m], tmsb_v, tb_sem)
        pltpu.async_copy(tlsb_ref.at[m], tlsb_v, tb_sem)

    def wait_tbl(m):
        pltpu.make_async_copy(tmsb_ref.at[m], tmsb_v, tb_sem).wait()
        pltpu.make_async_copy(tlsb_ref.at[m], tlsb_v, tb_sem).wait()

    start_tbl(0)
    pltpu.sync_copy(t256_ref, t256_v)

    def in_src(ci):
        return xp_ref.at[b, pl.ds(h0 + ci * _R, _R + 2)]

    def pv_src(ci):
        return out_ref.at[b, 0, pl.ds(h0 + ci * _R, _R)]

    def wr_pair(ci, x, m):
        if m < 2:
            return o_v[x].at[0], pv_src(ci)
        return o_v[x], out_ref.at[b, :, pl.ds(h0 + ci * _R, _R)]

    def start_in(ci, x):
        pltpu.async_copy(in_src(ci), in_v[x], in_sem[x])

    def start_pv(ci, x):
        pltpu.async_copy(pv_src(ci), pv_v[x], pv_sem[x])

    for m in range(3):
        offs = _MODE_OFFSETS[m]

        def step(ci, x, first, m=m, offs=offs):
            y = 1 - x
            iv, pv, ov = in_v[x], pv_v[x], o_v[x]
            wsrc, wdst = wr_pair(ci, x, m)
            pltpu.make_async_copy(in_src(ci), iv, in_sem[x]).wait()
            if m > 0:
                pltpu.make_async_copy(pv_src(ci), pv, pv_sem[x]).wait()
            if not first:
                pltpu.make_async_copy(wsrc, wdst, wr_sem[x]).wait()
            if m > 0:
                if isinstance(ci, int):
                    if ci + 1 < _CHUNKS:
                        start_pv(ci + 1, y)
                else:
                    @pl.when(ci < _CHUNKS - 1)
                    def _():
                        start_pv(ci + 1, y)

            @plsc.parallel_loop(0, _R * (_W // 16), 1, unroll=2)
            def it(i):
                r = i >> 5
                x0 = (i & 31) << 4
                p = [iv[r + dy, pl.ds(x0 + dx, 16)] for (dy, dx) in offs]
                ip = [q.astype(jnp.int32) for q in p]
                idx01 = (ip[0] & 0xF0) | (ip[1] >> 4)
                base12 = plsc.load_gather(t256_v, [idx01])
                msb3 = ip[3] >> 4
                flat = base12 + (ip[2] & 0xF0) + (ip[2] >> 4) + msb3
                wv = plsc.load_gather(tmsb_v, [flat])
                wvh = plsc.load_gather(tmsb_v, [flat + 1])
                idxl = (((ip[0] & 15) << 12) | ((ip[1] & 15) << 8)
                        | ((ip[2] & 15) << 4) | (ip[3] & 15))
                wl = plsc.load_gather(tlsb_v, [idxl])
                frac = p[3] * 0.0625 - msb3.astype(jnp.float32)
                fb = plsc.pack(frac, frac, format=plsc.PackFormat.INTERLEAVED)
                vb = _bf(wv)
                t = vb + (_bf(wvh) - vb) * fb + _bf(wl)
                if m > 0:
                    t = t + _bf(pv[r, pl.ds(x0, 16)])
                if m < 2:
                    ov[0, r, pl.ds(x0, 16)] = plsc.bitcast(t, jnp.float32)
                else:
                    c0, c1 = plsc.unpack(
                        t, format=plsc.PackFormat.INTERLEAVED)
                    ov[0, r, pl.ds(x0, 16)] = c0
                    ov[1, r, pl.ds(x0, 16)] = c1

            pltpu.async_copy(wsrc, wdst, wr_sem[x])
            if isinstance(ci, int):
                if ci + 2 < _CHUNKS:
                    start_in(ci + 2, x)
            else:
                @pl.when(ci < _CHUNKS - 2)
                def _():
                    start_in(ci + 2, x)

        start_in(0, 0)
        start_in(1, 1)
        if m > 0:
            start_pv(0, 0)
        wait_tbl(m)
        step(0, 0, first=True)
        step(1, 1, first=True)

        def pair(k, _):
            step(2 * k, 0, first=False)
            step(2 * k + 1, 1, first=False)
            return 0

        lax.fori_loop(1, _CHUNKS // 2, pair, 0)
        if m < 2:
            start_tbl(m + 1)
        pltpu.make_async_copy(*wr_pair(0, 0, m), wr_sem[0]).wait()
        pltpu.make_async_copy(*wr_pair(1, 1, m), wr_sem[1]).wait()


def _pack_pairs(t2):
    bits = jax.lax.bitcast_convert_type(
        t2.astype(jnp.bfloat16), jnp.uint16).astype(jnp.uint32)
    w = bits[..., 0] | (bits[..., 1] << 16)
    return jax.lax.bitcast_convert_type(w, jnp.int32)


@jax.jit
def kernel(x, lut_msb, lut_lsb, ref2index):
    xp = jnp.pad(x[:, 0], ((0, 0), (0, 2), (0, _WP - _W)), mode="edge")
    tmsb = _pack_pairs(
        lut_msb.reshape(3, _NC, _L * _L, 2)[:, :, :_MSTRIDE, :] * (1.0 / 3.0)
    ).reshape(3, _MSB_WORDS)
    tlsb = _pack_pairs(lut_lsb * (1.0 / 3.0))
    t256 = (ref2index[:16, :16].reshape(-1) * _MSTRIDE).astype(jnp.int32)

    mesh = plsc.VectorSubcoreMesh(core_axis_name="c", subcore_axis_name="s")
    run = pl.kernel(
        _body,
        out_type=jax.ShapeDtypeStruct((_B, 2, _H, _W), jnp.float32),
        mesh=mesh,
        compiler_params=pltpu.CompilerParams(
            use_tc_tiling_on_sc=False, needs_layout_passes=False),
        scratch_types=[
            pltpu.VMEM((256,), jnp.int32),
            pltpu.VMEM((_MSB_WORDS + _LSB_WORDS,), jnp.int32),
            [pltpu.VMEM((_R + 2, _WP), jnp.float32) for _ in range(2)],
            [pltpu.VMEM((_R, _W), jnp.float32) for _ in range(2)],
            [pltpu.VMEM((2, _R, _W), jnp.float32) for _ in range(2)],
            [pltpu.SemaphoreType.DMA for _ in range(2)],
            [pltpu.SemaphoreType.DMA for _ in range(2)],
            [pltpu.SemaphoreType.DMA for _ in range(2)],
            pltpu.SemaphoreType.DMA,
        ],
    )
    return run(xp, t256, tmsb, tlsb)
